# baseline (speedup 1.0000x reference)
import functools

import jax
import jax.numpy as jnp
from jax.experimental import pallas as pl
from jax.experimental.pallas import tpu as pltpu

S = 2048
E = 8
M = 768
DFF = 3072
ROWBLK = 128
N_ROWBLKS = S // ROWBLK
TOKBLK = 256


def _gating_kernel(x_ref, wg_ref, bg_ref, coeff_ref, cs2_ref):
    logits = jnp.dot(x_ref[...], wg_ref[...], preferred_element_type=jnp.float32)
    logits = logits + bg_ref[...]
    col = jax.lax.broadcasted_iota(jnp.int32, (S, E), 1)

    max1 = jnp.max(logits, axis=1, keepdims=True)
    idx1 = jnp.min(jnp.where(logits == max1, col, E), axis=1, keepdims=True)
    m1 = col == idx1

    neg_inf = jnp.float32(-jnp.inf)
    le1 = jnp.where(m1, neg_inf, logits)
    max2 = jnp.max(le1, axis=1, keepdims=True)
    idx2 = jnp.min(jnp.where(le1 == max2, col, E), axis=1, keepdims=True)
    m2 = col == idx2
    m1f = m1.astype(jnp.float32)
    m2f = m2.astype(jnp.float32)

    expx = jnp.exp(logits - max1)
    denom = jnp.sum(expx, axis=1, keepdims=True)
    g1 = jnp.sum(jnp.where(m1, expx, 0.0), axis=1, keepdims=True) / denom
    g2 = jnp.sum(jnp.where(m2, expx, 0.0), axis=1, keepdims=True) / denom

    def body(i, _):
        r0 = i * ROWBLK
        rr = jax.lax.broadcasted_iota(jnp.int32, (ROWBLK, S), 0)
        cc = jax.lax.broadcasted_iota(jnp.int32, (ROWBLK, S), 1)
        lt = (cc <= rr + r0).astype(jnp.float32)
        cs2_ref[pl.ds(r0, ROWBLK), :] = jnp.dot(
            lt, m2f, preferred_element_type=jnp.float32
        )
        return 0

    jax.lax.fori_loop(0, N_ROWBLKS, body, 0)

    count1 = jnp.sum(m1f, axis=0, keepdims=True)
    loc2 = (
        jnp.sum(cs2_ref[...] * m2f, axis=1, keepdims=True)
        - 1.0
        + jnp.sum(count1 * m2f, axis=1, keepdims=True)
    )
    valid2 = (loc2 < jnp.float32(S)).astype(jnp.float32)

    den = jnp.maximum(g1 + g2, jnp.float32(jnp.finfo(jnp.float32).eps))
    coeff_ref[...] = (g1 + g2 * valid2) / den


def _ffn_kernel(x_ref, w1_ref, b1_ref, w2_ref, b2_ref, coeff_ref, out_ref):
    h = jnp.dot(x_ref[...], w1_ref[...], preferred_element_type=jnp.float32)
    h = jnp.maximum(h + b1_ref[...], 0.0)
    y = jnp.dot(h.astype(jnp.bfloat16), w2_ref[...], preferred_element_type=jnp.float32)
    out_ref[...] = (y + b2_ref[...]) * coeff_ref[...]


@functools.partial(jax.jit, static_argnames=())
def kernel(inputs, Wg, bg, W1, b1, W2, b2):
    x = inputs.reshape(-1, M)

    coeff = pl.pallas_call(
        _gating_kernel,
        out_shape=jax.ShapeDtypeStruct((S, 1), jnp.float32),
        in_specs=[
            pl.BlockSpec((S, M), lambda: (0, 0)),
            pl.BlockSpec((M, E), lambda: (0, 0)),
            pl.BlockSpec((1, E), lambda: (0, 0)),
        ],
        out_specs=pl.BlockSpec((S, 1), lambda: (0, 0)),
        scratch_shapes=[pltpu.VMEM((S, E), jnp.float32)],
    )(x, Wg, bg.reshape(1, E))

    out = pl.pallas_call(
        _ffn_kernel,
        grid=(S // TOKBLK,),
        out_shape=jax.ShapeDtypeStruct((S, M), jnp.float32),
        in_specs=[
            pl.BlockSpec((TOKBLK, M), lambda i: (i, 0)),
            pl.BlockSpec((M, DFF), lambda i: (0, 0)),
            pl.BlockSpec((1, DFF), lambda i: (0, 0)),
            pl.BlockSpec((DFF, M), lambda i: (0, 0)),
            pl.BlockSpec((1, M), lambda i: (0, 0)),
            pl.BlockSpec((TOKBLK, 1), lambda i: (i, 0)),
        ],
        out_specs=pl.BlockSpec((TOKBLK, M), lambda i: (i, 0)),
    )(
        x.astype(jnp.bfloat16),
        W1.astype(jnp.bfloat16),
        b1.reshape(1, DFF),
        W2.astype(jnp.bfloat16),
        b2.reshape(1, M),
        coeff,
    )

    return out.reshape(inputs.shape)


# speedup vs baseline: 1.2411x; 1.2411x over previous
import functools

import jax
import jax.numpy as jnp
from jax.experimental import pallas as pl
from jax.experimental.pallas import tpu as pltpu

S = 2048
E = 8
M = 768
DFF = 3072
ROWBLK = 128
N_ROWBLKS = S // ROWBLK
TOKBLK = 256


def _fused_kernel(
    x_ref, wg_ref, bg_ref, w1_ref, b1_ref, w2_ref, b2_ref, out_ref, coeff_ref, cs2_ref
):
    step = pl.program_id(0)

    @pl.when(step == 0)
    def _gating():
        logits = jnp.dot(x_ref[...], wg_ref[...], preferred_element_type=jnp.float32)
        logits = logits + bg_ref[...]
        col = jax.lax.broadcasted_iota(jnp.int32, (S, E), 1)

        max1 = jnp.max(logits, axis=1, keepdims=True)
        idx1 = jnp.min(jnp.where(logits == max1, col, E), axis=1, keepdims=True)
        m1 = col == idx1

        neg_inf = jnp.float32(-jnp.inf)
        le1 = jnp.where(m1, neg_inf, logits)
        max2 = jnp.max(le1, axis=1, keepdims=True)
        idx2 = jnp.min(jnp.where(le1 == max2, col, E), axis=1, keepdims=True)
        m2 = col == idx2
        m1f = m1.astype(jnp.float32)
        m2f = m2.astype(jnp.float32)

        expx = jnp.exp(logits - max1)
        denom = jnp.sum(expx, axis=1, keepdims=True)
        g1 = jnp.sum(jnp.where(m1, expx, 0.0), axis=1, keepdims=True) / denom
        g2 = jnp.sum(jnp.where(m2, expx, 0.0), axis=1, keepdims=True) / denom

        def body(i, _):
            r0 = i * ROWBLK
            rr = jax.lax.broadcasted_iota(jnp.int32, (ROWBLK, S), 0)
            cc = jax.lax.broadcasted_iota(jnp.int32, (ROWBLK, S), 1)
            lt = (cc <= rr + r0).astype(jnp.float32)
            cs2_ref[pl.ds(r0, ROWBLK), :] = jnp.dot(
                lt, m2f, preferred_element_type=jnp.float32
            )
            return 0

        jax.lax.fori_loop(0, N_ROWBLKS, body, 0)

        count1 = jnp.sum(m1f, axis=0, keepdims=True)
        loc2 = (
            jnp.sum(cs2_ref[...] * m2f, axis=1, keepdims=True)
            - 1.0
            + jnp.sum(count1 * m2f, axis=1, keepdims=True)
        )
        valid2 = (loc2 < jnp.float32(S)).astype(jnp.float32)

        den = jnp.maximum(g1 + g2, jnp.float32(jnp.finfo(jnp.float32).eps))
        coeff_ref[...] = (g1 + g2 * valid2) / den

    @pl.when(step > 0)
    def _ffn():
        t0 = (step - 1) * TOKBLK
        xb = x_ref[pl.ds(t0, TOKBLK), :]
        h = jnp.dot(xb, w1_ref[...], preferred_element_type=jnp.float32)
        h = jnp.maximum(h + b1_ref[...], 0.0)
        y = jnp.dot(h, w2_ref[...], preferred_element_type=jnp.float32)
        out_ref[...] = (y + b2_ref[...]) * coeff_ref[pl.ds(t0, TOKBLK), :]


@functools.partial(jax.jit, static_argnames=())
def kernel(inputs, Wg, bg, W1, b1, W2, b2):
    x = inputs.reshape(-1, M)

    out = pl.pallas_call(
        _fused_kernel,
        grid=(1 + S // TOKBLK,),
        out_shape=jax.ShapeDtypeStruct((S, M), jnp.float32),
        in_specs=[
            pl.BlockSpec((S, M), lambda i: (0, 0)),
            pl.BlockSpec((M, E), lambda i: (0, 0)),
            pl.BlockSpec((1, E), lambda i: (0, 0)),
            pl.BlockSpec((M, DFF), lambda i: (0, 0)),
            pl.BlockSpec((1, DFF), lambda i: (0, 0)),
            pl.BlockSpec((DFF, M), lambda i: (0, 0)),
            pl.BlockSpec((1, M), lambda i: (0, 0)),
        ],
        out_specs=pl.BlockSpec(
            (TOKBLK, M), lambda i: (jnp.maximum(i - 1, 0), 0)
        ),
        scratch_shapes=[
            pltpu.VMEM((S, 1), jnp.float32),
            pltpu.VMEM((S, E), jnp.float32),
        ],
    )(x, Wg, bg.reshape(1, E), W1, b1.reshape(1, DFF), W2, b2.reshape(1, M))

    return out.reshape(inputs.shape)


# speedup vs baseline: 1.4213x; 1.1451x over previous
import functools

import jax
import jax.numpy as jnp
from jax.experimental import pallas as pl
from jax.experimental.pallas import tpu as pltpu

S = 2048
E = 8
M = 768
DFF = 3072
CUMCHUNK = 512
TOKBLK = 256


def _fused_kernel(
    x_ref, wg_ref, bg_ref, w1_ref, b1_ref, w2_ref, b2_ref, out_ref, coeff_ref
):
    step = pl.program_id(0)

    @pl.when(step == 0)
    def _gating():
        logits = jnp.dot(x_ref[...], wg_ref[...], preferred_element_type=jnp.float32)
        logits = logits + bg_ref[...]
        lt = jnp.transpose(logits)
        row = jax.lax.broadcasted_iota(jnp.int32, (E, S), 0)

        max1 = jnp.max(lt, axis=0, keepdims=True)
        idx1 = jnp.min(jnp.where(lt == max1, row, E), axis=0, keepdims=True)
        m1 = row == idx1

        neg_inf = jnp.float32(-jnp.inf)
        le1 = jnp.where(m1, neg_inf, lt)
        max2 = jnp.max(le1, axis=0, keepdims=True)
        idx2 = jnp.min(jnp.where(le1 == max2, row, E), axis=0, keepdims=True)
        m2 = row == idx2
        m1f = m1.astype(jnp.float32)
        m2f = m2.astype(jnp.float32)

        expx = jnp.exp(lt - max1)
        denom = jnp.sum(expx, axis=0, keepdims=True)
        g1 = jnp.sum(jnp.where(m1, expx, 0.0), axis=0, keepdims=True) / denom
        g2 = jnp.sum(jnp.where(m2, expx, 0.0), axis=0, keepdims=True) / denom

        cs2 = jnp.zeros((E, S), jnp.float32)
        for c in range(S // CUMCHUNK):
            t0 = c * CUMCHUNK
            rr = jax.lax.broadcasted_iota(jnp.int32, (CUMCHUNK, S), 0)
            cc = jax.lax.broadcasted_iota(jnp.int32, (CUMCHUNK, S), 1)
            ut = (rr + t0 <= cc).astype(jnp.float32)
            chunk = m2f[:, t0 : t0 + CUMCHUNK]
            cs2 = cs2 + jnp.dot(chunk, ut, preferred_element_type=jnp.float32)

        count1 = jnp.sum(m1f, axis=1, keepdims=True)
        loc2 = (
            jnp.sum(cs2 * m2f, axis=0, keepdims=True)
            - 1.0
            + jnp.sum(count1 * m2f, axis=0, keepdims=True)
        )
        valid2 = (loc2 < jnp.float32(S)).astype(jnp.float32)

        den = jnp.maximum(g1 + g2, jnp.float32(jnp.finfo(jnp.float32).eps))
        coeff_ref[...] = jnp.transpose((g1 + g2 * valid2) / den)

    @pl.when(step > 0)
    def _ffn():
        t0 = (step - 1) * TOKBLK
        xb = x_ref[pl.ds(t0, TOKBLK), :]
        h = jnp.dot(xb, w1_ref[...], preferred_element_type=jnp.float32)
        h = jnp.maximum(h + b1_ref[...], 0.0)
        y = jnp.dot(h, w2_ref[...], preferred_element_type=jnp.float32)
        out_ref[...] = (y + b2_ref[...]) * coeff_ref[pl.ds(t0, TOKBLK), :]


@functools.partial(jax.jit, static_argnames=())
def kernel(inputs, Wg, bg, W1, b1, W2, b2):
    x = inputs.reshape(-1, M)

    out = pl.pallas_call(
        _fused_kernel,
        grid=(1 + S // TOKBLK,),
        out_shape=jax.ShapeDtypeStruct((S, M), jnp.float32),
        in_specs=[
            pl.BlockSpec((S, M), lambda i: (0, 0)),
            pl.BlockSpec((M, E), lambda i: (0, 0)),
            pl.BlockSpec((1, E), lambda i: (0, 0)),
            pl.BlockSpec((M, DFF), lambda i: (0, 0)),
            pl.BlockSpec((1, DFF), lambda i: (0, 0)),
            pl.BlockSpec((DFF, M), lambda i: (0, 0)),
            pl.BlockSpec((1, M), lambda i: (0, 0)),
        ],
        out_specs=pl.BlockSpec(
            (TOKBLK, M), lambda i: (jnp.maximum(i - 1, 0), 0)
        ),
        scratch_shapes=[
            pltpu.VMEM((S, 1), jnp.float32),
        ],
    )(x, Wg, bg.reshape(1, E), W1, b1.reshape(1, DFF), W2, b2.reshape(1, M))

    return out.reshape(inputs.shape)
